# Initial kernel scaffold; baseline (speedup 1.0000x reference)
import jax
import jax.numpy as jnp
from jax.experimental import pallas as pl


def kernel(x, edge_index, edge_attr, W1, b1, W2, b2, W3, b3):
    raise NotImplementedError("write your pallas kernel here")


# speedup vs baseline: 5.6580x; 5.6580x over previous
import functools

import jax
import jax.numpy as jnp
from jax import lax
from jax.experimental import pallas as pl
from jax.experimental.pallas import tpu as pltpu
from jax.experimental.pallas import tpu_sc as plsc

N = 10000
E = 320000
D_NODE = 128
D_EDGE = 16
NEW_EDGE = 16
NEW_NODE = 8
FINAL_NODE = 4

NC = 2
NS = 16
NW = NC * NS
EW = E // NW
CHUNK = 80
NCHUNKS = EW // CHUNK
COPY_SUBS = 10
ROWS_PER_SUB = N // COPY_SUBS


def _node_proj(x, W1a, W1b, W2a):
    blk = 1000

    def body(x_ref, wa_ref, wb_ref, wq_ref, ps_ref, pd_ref, q_ref):
        xb = x_ref[...]
        ps_ref[...] = jnp.dot(xb, wa_ref[...], preferred_element_type=jnp.float32)
        pd_ref[...] = jnp.dot(xb, wb_ref[...], preferred_element_type=jnp.float32)
        q_ref[...] = jnp.dot(xb, wq_ref[...], preferred_element_type=jnp.float32)

    return pl.pallas_call(
        body,
        grid=(N // blk,),
        in_specs=[
            pl.BlockSpec((blk, D_NODE), lambda i: (i, 0)),
            pl.BlockSpec((D_NODE, NEW_EDGE), lambda i: (0, 0)),
            pl.BlockSpec((D_NODE, NEW_EDGE), lambda i: (0, 0)),
            pl.BlockSpec((D_NODE, NEW_NODE), lambda i: (0, 0)),
        ],
        out_specs=[
            pl.BlockSpec((blk, NEW_EDGE), lambda i: (i, 0)),
            pl.BlockSpec((blk, NEW_EDGE), lambda i: (i, 0)),
            pl.BlockSpec((blk, NEW_NODE), lambda i: (i, 0)),
        ],
        out_shape=[
            jax.ShapeDtypeStruct((N, NEW_EDGE), jnp.float32),
            jax.ShapeDtypeStruct((N, NEW_EDGE), jnp.float32),
            jax.ShapeDtypeStruct((N, NEW_NODE), jnp.float32),
        ],
    )(x, W1a, W1b, W2a)


def _edge_proj(edge8, Wbd, b1t):
    blk = 4000
    rows = E // 8

    def body(a_ref, w_ref, b_ref, o_ref):
        o_ref[...] = (
            jnp.dot(a_ref[...], w_ref[...], preferred_element_type=jnp.float32)
            + b_ref[...]
        )

    return pl.pallas_call(
        body,
        grid=(rows // blk,),
        in_specs=[
            pl.BlockSpec((blk, 128), lambda i: (i, 0)),
            pl.BlockSpec((128, 128), lambda i: (0, 0)),
            pl.BlockSpec((1, 128), lambda i: (0, 0)),
        ],
        out_specs=pl.BlockSpec((blk, 128), lambda i: (i, 0)),
        out_shape=jax.ShapeDtypeStruct((rows, 128), jnp.float32),
    )(edge8, Wbd, b1t)


def _edge_aggregate(src, dst, p_src, p_dst, t):
    mesh = plsc.VectorSubcoreMesh(core_axis_name="c", subcore_axis_name="s")

    @functools.partial(
        pl.kernel,
        mesh=mesh,
        compiler_params=pltpu.CompilerParams(use_tc_tiling_on_sc=False),
        out_type=jax.ShapeDtypeStruct((NC, N, NEW_EDGE), jnp.float32),
        scratch_types=[
            pltpu.VMEM((CHUNK,), jnp.int32),
            pltpu.VMEM((CHUNK,), jnp.int32),
            pltpu.VMEM((CHUNK, NEW_EDGE), jnp.float32),
            pltpu.VMEM((CHUNK, NEW_EDGE), jnp.float32),
            pltpu.VMEM((CHUNK, NEW_EDGE), jnp.float32),
            pltpu.VMEM((ROWS_PER_SUB, NEW_EDGE), jnp.float32),
            pltpu.VMEM_SHARED((N, NEW_EDGE), jnp.float32),
            pltpu.SemaphoreType.DMA,
            pltpu.SemaphoreType.DMA,
        ],
    )
    def body(src_hbm, dst_hbm, ps_hbm, pd_hbm, t_hbm, out_hbm,
             sidx, didx, srows, drows, trows, zbuf, agg, sem1, sem2):
        c = lax.axis_index("c")
        s = lax.axis_index("s")
        wid = s * NC + c

        @pl.when(s < COPY_SUBS)
        def _():
            def zero_row(j, _):
                zbuf[j] = jnp.zeros((NEW_EDGE,), jnp.float32)
                return 0

            lax.fori_loop(0, ROWS_PER_SUB, zero_row, 0)
            pltpu.sync_copy(zbuf, agg.at[pl.ds(s * ROWS_PER_SUB, ROWS_PER_SUB)])

        plsc.subcore_barrier()

        def chunk_body(ci, _):
            ebase = wid * EW + ci * CHUNK
            pltpu.sync_copy(src_hbm.at[pl.ds(ebase, CHUNK)], sidx)
            pltpu.sync_copy(dst_hbm.at[pl.ds(ebase, CHUNK)], didx)
            cp1 = pltpu.async_copy(ps_hbm.at[sidx], srows, sem1)
            cp2 = pltpu.async_copy(pd_hbm.at[didx], drows, sem2)
            pltpu.sync_copy(t_hbm.at[pl.ds(ebase, CHUNK)], trows)
            cp1.wait()
            cp2.wait()

            def edge_body(j, _):
                v = srows[j] + drows[j] + trows[j]
                srows[j] = jnp.maximum(v, 0.0)
                return 0

            lax.fori_loop(0, CHUNK, edge_body, 0)
            pltpu.sync_copy(srows, agg.at[didx], add=True)
            return 0

        lax.fori_loop(0, NCHUNKS, chunk_body, 0)
        plsc.subcore_barrier()

        @pl.when(s < COPY_SUBS)
        def _():
            base = s * ROWS_PER_SUB
            pltpu.sync_copy(
                agg.at[pl.ds(base, ROWS_PER_SUB)],
                out_hbm.at[c, pl.ds(base, ROWS_PER_SUB)],
            )

    return body(src, dst, p_src, p_dst, t)


def _node_update(agg0, agg1, q, W2b, b2, W3, b3):
    blk = 1000

    def body(a0_ref, a1_ref, q_ref, w2_ref, b2_ref, w3_ref, b3_ref, o_ref):
        agg = a0_ref[...] + a1_ref[...]
        h = jnp.dot(agg, w2_ref[...], preferred_element_type=jnp.float32)
        h = jnp.maximum(h + q_ref[...] + b2_ref[...], 0.0)
        o_ref[...] = (
            jnp.dot(h, w3_ref[...], preferred_element_type=jnp.float32)
            + b3_ref[...]
        )

    return pl.pallas_call(
        body,
        grid=(N // blk,),
        in_specs=[
            pl.BlockSpec((blk, NEW_EDGE), lambda i: (i, 0)),
            pl.BlockSpec((blk, NEW_EDGE), lambda i: (i, 0)),
            pl.BlockSpec((blk, NEW_NODE), lambda i: (i, 0)),
            pl.BlockSpec((NEW_EDGE, NEW_NODE), lambda i: (0, 0)),
            pl.BlockSpec((1, NEW_NODE), lambda i: (0, 0)),
            pl.BlockSpec((NEW_NODE, FINAL_NODE), lambda i: (0, 0)),
            pl.BlockSpec((1, FINAL_NODE), lambda i: (0, 0)),
        ],
        out_specs=pl.BlockSpec((blk, FINAL_NODE), lambda i: (i, 0)),
        out_shape=jax.ShapeDtypeStruct((N, FINAL_NODE), jnp.float32),
    )(agg0, agg1, q, W2b, b2, W3, b3)


def kernel(x, edge_index, edge_attr, W1, b1, W2, b2, W3, b3):
    W1a = W1[:D_NODE]
    W1b = W1[D_NODE:2 * D_NODE]
    W1c = W1[2 * D_NODE:]
    W2a = W2[:D_NODE]
    W2b = W2[D_NODE:]

    p_src, p_dst, q = _node_proj(x, W1a, W1b, W2a)

    edge8 = edge_attr.reshape(E // 8, 128)
    Wbd = jnp.kron(jnp.eye(8, dtype=jnp.float32), W1c)
    b1t = jnp.tile(b1, 8).reshape(1, 128)
    t8 = _edge_proj(edge8, Wbd, b1t)
    t = t8.reshape(E, NEW_EDGE)

    src = edge_index[0]
    dst = edge_index[1]
    aggs = _edge_aggregate(src, dst, p_src, p_dst, t)

    out = _node_update(aggs[0], aggs[1], q, W2b, b2.reshape(1, -1),
                       W3, b3.reshape(1, -1))
    return out


# speedup vs baseline: 7.4606x; 1.3186x over previous
import functools

import jax
import jax.numpy as jnp
from jax import lax
from jax.experimental import pallas as pl
from jax.experimental.pallas import tpu as pltpu
from jax.experimental.pallas import tpu_sc as plsc

N = 10000
E = 320000
D_NODE = 128
D_EDGE = 16
NEW_EDGE = 16
NEW_NODE = 8
FINAL_NODE = 4

NC = 2
NS = 16
NW = NC * NS
EW = E // NW
CHUNK = 80
NCHUNKS = EW // CHUNK
COPY_SUBS = 10
ROWS_PER_SUB = N // COPY_SUBS


def _node_proj(x, W1a, W1b, W2a):
    blk = 1000

    def body(x_ref, wa_ref, wb_ref, wq_ref, ps_ref, pd_ref, q_ref):
        xb = x_ref[...]
        ps_ref[...] = jnp.dot(xb, wa_ref[...], preferred_element_type=jnp.float32)
        pd_ref[...] = jnp.dot(xb, wb_ref[...], preferred_element_type=jnp.float32)
        q_ref[...] = jnp.dot(xb, wq_ref[...], preferred_element_type=jnp.float32)

    return pl.pallas_call(
        body,
        grid=(N // blk,),
        in_specs=[
            pl.BlockSpec((blk, D_NODE), lambda i: (i, 0)),
            pl.BlockSpec((D_NODE, NEW_EDGE), lambda i: (0, 0)),
            pl.BlockSpec((D_NODE, NEW_EDGE), lambda i: (0, 0)),
            pl.BlockSpec((D_NODE, NEW_NODE), lambda i: (0, 0)),
        ],
        out_specs=[
            pl.BlockSpec((blk, NEW_EDGE), lambda i: (i, 0)),
            pl.BlockSpec((blk, NEW_EDGE), lambda i: (i, 0)),
            pl.BlockSpec((blk, NEW_NODE), lambda i: (i, 0)),
        ],
        out_shape=[
            jax.ShapeDtypeStruct((N, NEW_EDGE), jnp.float32),
            jax.ShapeDtypeStruct((N, NEW_EDGE), jnp.float32),
            jax.ShapeDtypeStruct((N, NEW_NODE), jnp.float32),
        ],
    )(x, W1a, W1b, W2a)


def _edge_proj(edge8, Wbd, b1t):
    blk = 4000
    rows = E // 8

    def body(a_ref, w_ref, b_ref, o_ref):
        o_ref[...] = (
            jnp.dot(a_ref[...], w_ref[...], preferred_element_type=jnp.float32)
            + b_ref[...]
        )

    return pl.pallas_call(
        body,
        grid=(rows // blk,),
        in_specs=[
            pl.BlockSpec((blk, 128), lambda i: (i, 0)),
            pl.BlockSpec((128, 128), lambda i: (0, 0)),
            pl.BlockSpec((1, 128), lambda i: (0, 0)),
        ],
        out_specs=pl.BlockSpec((blk, 128), lambda i: (i, 0)),
        out_shape=jax.ShapeDtypeStruct((rows, 128), jnp.float32),
    )(edge8, Wbd, b1t)


def _edge_aggregate(src, dst, p_src, p_dst, t_flat):
    mesh = plsc.VectorSubcoreMesh(core_axis_name="c", subcore_axis_name="s")

    @functools.partial(
        pl.kernel,
        mesh=mesh,
        compiler_params=pltpu.CompilerParams(use_tc_tiling_on_sc=False),
        out_type=jax.ShapeDtypeStruct((NC, N, NEW_EDGE), jnp.float32),
        scratch_types=[
            pltpu.VMEM((2, CHUNK), jnp.int32),
            pltpu.VMEM((2, CHUNK), jnp.int32),
            pltpu.VMEM((2, CHUNK, NEW_EDGE), jnp.float32),
            pltpu.VMEM((2, CHUNK, NEW_EDGE), jnp.float32),
            pltpu.VMEM((2, CHUNK * NEW_EDGE), jnp.float32),
            pltpu.VMEM((ROWS_PER_SUB, NEW_EDGE), jnp.float32),
            pltpu.VMEM_SHARED((N, NEW_EDGE), jnp.float32),
            pltpu.SemaphoreType.DMA,
            pltpu.SemaphoreType.DMA,
            pltpu.SemaphoreType.DMA,
            pltpu.SemaphoreType.DMA,
        ],
    )
    def body(src_hbm, dst_hbm, ps_hbm, pd_hbm, t_hbm, out_hbm,
             sidx, didx, srows, drows, trows, zbuf, agg,
             sem_i0, sem_i1, sem_g0, sem_g1):
        c = lax.axis_index("c")
        s = lax.axis_index("s")
        wid = s * NC + c
        sem_i = (sem_i0, sem_i1)
        sem_g = (sem_g0, sem_g1)

        @pl.when(s < COPY_SUBS)
        def _():
            def zero_row(j, _):
                zbuf[j] = jnp.zeros((NEW_EDGE,), jnp.float32)
                return 0

            lax.fori_loop(0, ROWS_PER_SUB, zero_row, 0)
            pltpu.sync_copy(zbuf, agg.at[pl.ds(s * ROWS_PER_SUB, ROWS_PER_SUB)])

        plsc.subcore_barrier()

        def issue_idx(ci, b):
            ebase = wid * EW + ci * CHUNK
            pltpu.async_copy(src_hbm.at[pl.ds(ebase, CHUNK)], sidx.at[b],
                             sem_i[b])
            pltpu.async_copy(dst_hbm.at[pl.ds(ebase, CHUNK)], didx.at[b],
                             sem_i[b])
            pltpu.async_copy(
                t_hbm.at[pl.ds(ebase * NEW_EDGE, CHUNK * NEW_EDGE)],
                trows.at[b], sem_i[b])

        def wait_idx(ci, b):
            ebase = wid * EW + ci * CHUNK
            pltpu.make_async_copy(src_hbm.at[pl.ds(ebase, CHUNK)],
                                  sidx.at[b], sem_i[b]).wait()
            pltpu.make_async_copy(dst_hbm.at[pl.ds(ebase, CHUNK)],
                                  didx.at[b], sem_i[b]).wait()
            pltpu.make_async_copy(
                t_hbm.at[pl.ds(ebase * NEW_EDGE, CHUNK * NEW_EDGE)],
                trows.at[b], sem_i[b]).wait()

        def issue_gather(b):
            pltpu.async_copy(ps_hbm.at[sidx.at[b]], srows.at[b], sem_g[b])
            pltpu.async_copy(pd_hbm.at[didx.at[b]], drows.at[b], sem_g[b])

        def wait_gather(b):
            pltpu.make_async_copy(ps_hbm.at[sidx.at[b]], srows.at[b],
                                  sem_g[b]).wait()
            pltpu.make_async_copy(pd_hbm.at[didx.at[b]], drows.at[b],
                                  sem_g[b]).wait()

        def process(ci, b):
            @pl.when(ci + 1 < NCHUNKS)
            def _():
                wait_idx(ci + 1, 1 - b)
                issue_gather(1 - b)

            wait_gather(b)
            sr = srows.at[b]
            dr = drows.at[b]
            tr = trows.at[b]

            def edge_body(j, _):
                v = sr[j] + dr[j] + tr[pl.ds(j * NEW_EDGE, NEW_EDGE)]
                sr[j] = jnp.maximum(v, 0.0)
                return 0

            lax.fori_loop(0, CHUNK, edge_body, 0)
            pltpu.sync_copy(sr, agg.at[didx.at[b]], add=True)

            @pl.when(ci + 2 < NCHUNKS)
            def _():
                issue_idx(ci + 2, b)

        issue_idx(0, 0)
        issue_idx(1, 1)
        wait_idx(0, 0)
        issue_gather(0)

        def pair_body(i, _):
            ci = 2 * i
            process(ci, 0)
            process(ci + 1, 1)
            return 0

        lax.fori_loop(0, NCHUNKS // 2, pair_body, 0)
        process(NCHUNKS - 1, 0)
        plsc.subcore_barrier()

        @pl.when(s < COPY_SUBS)
        def _():
            base = s * ROWS_PER_SUB
            pltpu.sync_copy(
                agg.at[pl.ds(base, ROWS_PER_SUB)],
                out_hbm.at[c, pl.ds(base, ROWS_PER_SUB)],
            )

    return body(src, dst, p_src, p_dst, t_flat)


def _node_update(agg0, agg1, q, W2b, b2, W3, b3):
    blk = 1000

    def body(a0_ref, a1_ref, q_ref, w2_ref, b2_ref, w3_ref, b3_ref, o_ref):
        agg = a0_ref[...] + a1_ref[...]
        h = jnp.dot(agg, w2_ref[...], preferred_element_type=jnp.float32)
        h = jnp.maximum(h + q_ref[...] + b2_ref[...], 0.0)
        o_ref[...] = (
            jnp.dot(h, w3_ref[...], preferred_element_type=jnp.float32)
            + b3_ref[...]
        )

    return pl.pallas_call(
        body,
        grid=(N // blk,),
        in_specs=[
            pl.BlockSpec((blk, NEW_EDGE), lambda i: (i, 0)),
            pl.BlockSpec((blk, NEW_EDGE), lambda i: (i, 0)),
            pl.BlockSpec((blk, NEW_NODE), lambda i: (i, 0)),
            pl.BlockSpec((NEW_EDGE, NEW_NODE), lambda i: (0, 0)),
            pl.BlockSpec((1, NEW_NODE), lambda i: (0, 0)),
            pl.BlockSpec((NEW_NODE, FINAL_NODE), lambda i: (0, 0)),
            pl.BlockSpec((1, FINAL_NODE), lambda i: (0, 0)),
        ],
        out_specs=pl.BlockSpec((blk, FINAL_NODE), lambda i: (i, 0)),
        out_shape=jax.ShapeDtypeStruct((N, FINAL_NODE), jnp.float32),
    )(agg0, agg1, q, W2b, b2, W3, b3)


def kernel(x, edge_index, edge_attr, W1, b1, W2, b2, W3, b3):
    W1a = W1[:D_NODE]
    W1b = W1[D_NODE:2 * D_NODE]
    W1c = W1[2 * D_NODE:]
    W2a = W2[:D_NODE]
    W2b = W2[D_NODE:]

    p_src, p_dst, q = _node_proj(x, W1a, W1b, W2a)

    edge8 = edge_attr.reshape(E // 8, 128)
    Wbd = jnp.kron(jnp.eye(8, dtype=jnp.float32), W1c)
    b1t = jnp.tile(b1, 8).reshape(1, 128)
    t8 = _edge_proj(edge8, Wbd, b1t)
    t_flat = t8.reshape(E * NEW_EDGE)

    src = edge_index[0]
    dst = edge_index[1]
    aggs = _edge_aggregate(src, dst, p_src, p_dst, t_flat)

    out = _node_update(aggs[0], aggs[1], q, W2b, b2.reshape(1, -1),
                       W3, b3.reshape(1, -1))
    return out


# speedup vs baseline: 8.2688x; 1.1083x over previous
import functools

import jax
import jax.numpy as jnp
from jax import lax
from jax.experimental import pallas as pl
from jax.experimental.pallas import tpu as pltpu
from jax.experimental.pallas import tpu_sc as plsc

N = 10000
E = 320000
D_NODE = 128
D_EDGE = 16
NEW_EDGE = 16
NEW_NODE = 8
FINAL_NODE = 4

NC = 2
NS = 16
NW = NC * NS
EW = E // NW
CHUNK = 128
NCHUNKS = EW // CHUNK
TAIL = EW - NCHUNKS * CHUNK
COPY_SUBS = 10
ROWS_PER_SUB = N // COPY_SUBS


def _prologue_proj(x, W1a, W1b, W2a, edge8, Wbd, b1t):
    nblk = 1000
    eblk = 4000
    erows = E // 8

    def body(x_ref, wa_ref, wb_ref, wq_ref, a_ref, w_ref, b_ref,
             ps_ref, pd_ref, q_ref, t_ref):
        xb = x_ref[...]
        ps_ref[...] = jnp.dot(xb, wa_ref[...], preferred_element_type=jnp.float32)
        pd_ref[...] = jnp.dot(xb, wb_ref[...], preferred_element_type=jnp.float32)
        q_ref[...] = jnp.dot(xb, wq_ref[...], preferred_element_type=jnp.float32)
        t_ref[...] = (
            jnp.dot(a_ref[...], w_ref[...], preferred_element_type=jnp.float32)
            + b_ref[...]
        )

    return pl.pallas_call(
        body,
        grid=(N // nblk,),
        in_specs=[
            pl.BlockSpec((nblk, D_NODE), lambda i: (i, 0)),
            pl.BlockSpec((D_NODE, NEW_EDGE), lambda i: (0, 0)),
            pl.BlockSpec((D_NODE, NEW_EDGE), lambda i: (0, 0)),
            pl.BlockSpec((D_NODE, NEW_NODE), lambda i: (0, 0)),
            pl.BlockSpec((eblk, 128), lambda i: (i, 0)),
            pl.BlockSpec((128, 128), lambda i: (0, 0)),
            pl.BlockSpec((1, 128), lambda i: (0, 0)),
        ],
        out_specs=[
            pl.BlockSpec((nblk, NEW_EDGE), lambda i: (i, 0)),
            pl.BlockSpec((nblk, NEW_EDGE), lambda i: (i, 0)),
            pl.BlockSpec((nblk, NEW_NODE), lambda i: (i, 0)),
            pl.BlockSpec((eblk, 128), lambda i: (i, 0)),
        ],
        out_shape=[
            jax.ShapeDtypeStruct((N, NEW_EDGE), jnp.float32),
            jax.ShapeDtypeStruct((N, NEW_EDGE), jnp.float32),
            jax.ShapeDtypeStruct((N, NEW_NODE), jnp.float32),
            jax.ShapeDtypeStruct((erows, 128), jnp.float32),
        ],
    )(x, W1a, W1b, W2a, edge8, Wbd, b1t)


def _edge_aggregate(src, dst, p_src, p_dst, t_flat):
    mesh = plsc.VectorSubcoreMesh(core_axis_name="c", subcore_axis_name="s")

    @functools.partial(
        pl.kernel,
        mesh=mesh,
        compiler_params=pltpu.CompilerParams(use_tc_tiling_on_sc=False),
        out_type=jax.ShapeDtypeStruct((NC, N, NEW_EDGE), jnp.float32),
        scratch_types=[
            pltpu.VMEM((2, CHUNK), jnp.int32),
            pltpu.VMEM((2, CHUNK), jnp.int32),
            pltpu.VMEM((2, CHUNK, NEW_EDGE), jnp.float32),
            pltpu.VMEM((2, CHUNK, NEW_EDGE), jnp.float32),
            pltpu.VMEM((2, CHUNK * NEW_EDGE), jnp.float32),
            pltpu.VMEM((ROWS_PER_SUB, NEW_EDGE), jnp.float32),
            pltpu.VMEM_SHARED((N, NEW_EDGE), jnp.float32),
            pltpu.VMEM((TAIL,), jnp.int32),
            pltpu.VMEM((TAIL,), jnp.int32),
            pltpu.VMEM((TAIL, NEW_EDGE), jnp.float32),
            pltpu.VMEM((TAIL, NEW_EDGE), jnp.float32),
            pltpu.VMEM((TAIL * NEW_EDGE,), jnp.float32),
            pltpu.SemaphoreType.DMA,
            pltpu.SemaphoreType.DMA,
            pltpu.SemaphoreType.DMA,
            pltpu.SemaphoreType.DMA,
        ],
    )
    def body(src_hbm, dst_hbm, ps_hbm, pd_hbm, t_hbm, out_hbm,
             sidx, didx, srows, drows, trows, zbuf, agg,
             tsidx, tdidx, tsrows, tdrows, ttrows,
             sem_i0, sem_i1, sem_g0, sem_g1):
        c = lax.axis_index("c")
        s = lax.axis_index("s")
        wid = s * NC + c
        sem_i = (sem_i0, sem_i1)
        sem_g = (sem_g0, sem_g1)

        @pl.when(s < COPY_SUBS)
        def _():
            def zero_row(j, _):
                zbuf[j] = jnp.zeros((NEW_EDGE,), jnp.float32)
                return 0

            lax.fori_loop(0, ROWS_PER_SUB, zero_row, 0)
            pltpu.sync_copy(zbuf, agg.at[pl.ds(s * ROWS_PER_SUB, ROWS_PER_SUB)])

        plsc.subcore_barrier()

        def issue_idx(ci, b):
            ebase = wid * EW + ci * CHUNK
            pltpu.async_copy(src_hbm.at[pl.ds(ebase, CHUNK)], sidx.at[b],
                             sem_i[b])
            pltpu.async_copy(dst_hbm.at[pl.ds(ebase, CHUNK)], didx.at[b],
                             sem_i[b])
            pltpu.async_copy(
                t_hbm.at[pl.ds(ebase * NEW_EDGE, CHUNK * NEW_EDGE)],
                trows.at[b], sem_i[b])

        def wait_idx(ci, b):
            ebase = wid * EW + ci * CHUNK
            pltpu.make_async_copy(src_hbm.at[pl.ds(ebase, CHUNK)],
                                  sidx.at[b], sem_i[b]).wait()
            pltpu.make_async_copy(dst_hbm.at[pl.ds(ebase, CHUNK)],
                                  didx.at[b], sem_i[b]).wait()
            pltpu.make_async_copy(
                t_hbm.at[pl.ds(ebase * NEW_EDGE, CHUNK * NEW_EDGE)],
                trows.at[b], sem_i[b]).wait()

        def issue_gather(b):
            pltpu.async_copy(ps_hbm.at[sidx.at[b]], srows.at[b], sem_g[b])
            pltpu.async_copy(pd_hbm.at[didx.at[b]], drows.at[b], sem_g[b])

        def wait_gather(b):
            pltpu.make_async_copy(ps_hbm.at[sidx.at[b]], srows.at[b],
                                  sem_g[b]).wait()
            pltpu.make_async_copy(pd_hbm.at[didx.at[b]], drows.at[b],
                                  sem_g[b]).wait()

        def process(ci, b):
            @pl.when(ci + 1 < NCHUNKS)
            def _():
                wait_idx(ci + 1, 1 - b)
                issue_gather(1 - b)

            wait_gather(b)
            sr = srows.at[b]
            dr = drows.at[b]
            tr = trows.at[b]

            def edge_body(j, _):
                v = sr[j] + dr[j] + tr[pl.ds(j * NEW_EDGE, NEW_EDGE)]
                sr[j] = jnp.maximum(v, 0.0)
                return 0

            lax.fori_loop(0, CHUNK, edge_body, 0)
            pltpu.sync_copy(sr, agg.at[didx.at[b]], add=True)

            @pl.when(ci + 2 < NCHUNKS)
            def _():
                issue_idx(ci + 2, b)

        issue_idx(0, 0)
        issue_idx(1, 1)
        wait_idx(0, 0)
        issue_gather(0)

        def pair_body(i, _):
            ci = 2 * i
            process(ci, 0)
            process(ci + 1, 1)
            return 0

        lax.fori_loop(0, NCHUNKS // 2, pair_body, 0)
        if NCHUNKS % 2:
            process(NCHUNKS - 1, 0)

        if TAIL:
            tbase = wid * EW + NCHUNKS * CHUNK
            pltpu.sync_copy(src_hbm.at[pl.ds(tbase, TAIL)], tsidx)
            pltpu.sync_copy(dst_hbm.at[pl.ds(tbase, TAIL)], tdidx)
            cp1 = pltpu.async_copy(ps_hbm.at[tsidx], tsrows, sem_g0)
            cp2 = pltpu.async_copy(pd_hbm.at[tdidx], tdrows, sem_g1)
            pltpu.sync_copy(
                t_hbm.at[pl.ds(tbase * NEW_EDGE, TAIL * NEW_EDGE)], ttrows)
            cp1.wait()
            cp2.wait()

            def tail_body(j, _):
                v = tsrows[j] + tdrows[j] + ttrows[pl.ds(j * NEW_EDGE, NEW_EDGE)]
                tsrows[j] = jnp.maximum(v, 0.0)
                return 0

            lax.fori_loop(0, TAIL, tail_body, 0)
            pltpu.sync_copy(tsrows, agg.at[tdidx], add=True)

        plsc.subcore_barrier()

        @pl.when(s < COPY_SUBS)
        def _():
            base = s * ROWS_PER_SUB
            pltpu.sync_copy(
                agg.at[pl.ds(base, ROWS_PER_SUB)],
                out_hbm.at[c, pl.ds(base, ROWS_PER_SUB)],
            )

    return body(src, dst, p_src, p_dst, t_flat)


def _node_update(agg0, agg1, q, W2b, b2, W3, b3):
    blk = 1000

    def body(a0_ref, a1_ref, q_ref, w2_ref, b2_ref, w3_ref, b3_ref, o_ref):
        agg = a0_ref[...] + a1_ref[...]
        h = jnp.dot(agg, w2_ref[...], preferred_element_type=jnp.float32)
        h = jnp.maximum(h + q_ref[...] + b2_ref[...], 0.0)
        o_ref[...] = (
            jnp.dot(h, w3_ref[...], preferred_element_type=jnp.float32)
            + b3_ref[...]
        )

    return pl.pallas_call(
        body,
        grid=(N // blk,),
        in_specs=[
            pl.BlockSpec((blk, NEW_EDGE), lambda i: (i, 0)),
            pl.BlockSpec((blk, NEW_EDGE), lambda i: (i, 0)),
            pl.BlockSpec((blk, NEW_NODE), lambda i: (i, 0)),
            pl.BlockSpec((NEW_EDGE, NEW_NODE), lambda i: (0, 0)),
            pl.BlockSpec((1, NEW_NODE), lambda i: (0, 0)),
            pl.BlockSpec((NEW_NODE, FINAL_NODE), lambda i: (0, 0)),
            pl.BlockSpec((1, FINAL_NODE), lambda i: (0, 0)),
        ],
        out_specs=pl.BlockSpec((blk, FINAL_NODE), lambda i: (i, 0)),
        out_shape=jax.ShapeDtypeStruct((N, FINAL_NODE), jnp.float32),
    )(agg0, agg1, q, W2b, b2, W3, b3)


def kernel(x, edge_index, edge_attr, W1, b1, W2, b2, W3, b3):
    W1a = W1[:D_NODE]
    W1b = W1[D_NODE:2 * D_NODE]
    W1c = W1[2 * D_NODE:]
    W2a = W2[:D_NODE]
    W2b = W2[D_NODE:]

    edge8 = edge_attr.reshape(E // 8, 128)
    Wbd = jnp.kron(jnp.eye(8, dtype=jnp.float32), W1c)
    b1t = jnp.tile(b1, 8).reshape(1, 128)
    p_src, p_dst, q, t8 = _prologue_proj(x, W1a, W1b, W2a, edge8, Wbd, b1t)
    t_flat = t8.reshape(E * NEW_EDGE)

    src = edge_index[0]
    dst = edge_index[1]
    aggs = _edge_aggregate(src, dst, p_src, p_dst, t_flat)

    out = _node_update(aggs[0], aggs[1], q, W2b, b2.reshape(1, -1),
                       W3, b3.reshape(1, -1))
    return out


# speedup vs baseline: 8.6671x; 1.0482x over previous
import functools

import jax
import jax.numpy as jnp
from jax import lax
from jax.experimental import pallas as pl
from jax.experimental.pallas import tpu as pltpu
from jax.experimental.pallas import tpu_sc as plsc

N = 10000
E = 320000
D_NODE = 128
D_EDGE = 16
NEW_EDGE = 16
NEW_NODE = 8
FINAL_NODE = 4

NC = 2
NS = 16
NW = NC * NS
EW = E // NW
CHUNK = 128
NCHUNKS = EW // CHUNK
TAIL = EW - NCHUNKS * CHUNK
COPY_SUBS = 10
ROWS_PER_SUB = N // COPY_SUBS


def _prologue_proj(x, W1a, W1b, W2a, edge8, Wbd, b1t):
    nblk = 1000
    eblk = 4000
    erows = E // 8

    def body(x_ref, wa_ref, wb_ref, wq_ref, a_ref, w_ref, b_ref,
             ps_ref, pd_ref, q_ref, t_ref):
        xb = x_ref[...]
        ps_ref[...] = jnp.dot(xb, wa_ref[...], preferred_element_type=jnp.float32)
        pd_ref[...] = jnp.dot(xb, wb_ref[...], preferred_element_type=jnp.float32)
        q_ref[...] = jnp.dot(xb, wq_ref[...], preferred_element_type=jnp.float32)
        t_ref[...] = (
            jnp.dot(a_ref[...], w_ref[...], preferred_element_type=jnp.float32)
            + b_ref[...]
        )

    return pl.pallas_call(
        body,
        grid=(N // nblk,),
        in_specs=[
            pl.BlockSpec((nblk, D_NODE), lambda i: (i, 0)),
            pl.BlockSpec((D_NODE, NEW_EDGE), lambda i: (0, 0)),
            pl.BlockSpec((D_NODE, NEW_EDGE), lambda i: (0, 0)),
            pl.BlockSpec((D_NODE, NEW_NODE), lambda i: (0, 0)),
            pl.BlockSpec((eblk, 128), lambda i: (i, 0)),
            pl.BlockSpec((128, 128), lambda i: (0, 0)),
            pl.BlockSpec((1, 128), lambda i: (0, 0)),
        ],
        out_specs=[
            pl.BlockSpec((nblk, NEW_EDGE), lambda i: (i, 0)),
            pl.BlockSpec((nblk, NEW_EDGE), lambda i: (i, 0)),
            pl.BlockSpec((nblk, NEW_NODE), lambda i: (i, 0)),
            pl.BlockSpec((eblk, 128), lambda i: (i, 0)),
        ],
        out_shape=[
            jax.ShapeDtypeStruct((N, NEW_EDGE), jnp.float32),
            jax.ShapeDtypeStruct((N, NEW_EDGE), jnp.float32),
            jax.ShapeDtypeStruct((N, NEW_NODE), jnp.float32),
            jax.ShapeDtypeStruct((erows, 128), jnp.float32),
        ],
    )(x, W1a, W1b, W2a, edge8, Wbd, b1t)


def _edge_aggregate(src, dst, p_src, p_dst, t8):
    mesh = plsc.VectorSubcoreMesh(core_axis_name="c", subcore_axis_name="s")

    @functools.partial(
        pl.kernel,
        mesh=mesh,
        compiler_params=pltpu.CompilerParams(use_tc_tiling_on_sc=False),
        out_type=jax.ShapeDtypeStruct((NC, N, NEW_EDGE), jnp.float32),
        scratch_types=[
            pltpu.VMEM((2, CHUNK), jnp.int32),
            pltpu.VMEM((2, CHUNK), jnp.int32),
            pltpu.VMEM((2, CHUNK, NEW_EDGE), jnp.float32),
            pltpu.VMEM((2, CHUNK, NEW_EDGE), jnp.float32),
            pltpu.VMEM((2, CHUNK * NEW_EDGE // 128, 128), jnp.float32),
            pltpu.VMEM((ROWS_PER_SUB, NEW_EDGE), jnp.float32),
            pltpu.VMEM_SHARED((N, NEW_EDGE), jnp.float32),
            pltpu.VMEM((TAIL,), jnp.int32),
            pltpu.VMEM((TAIL,), jnp.int32),
            pltpu.VMEM((TAIL, NEW_EDGE), jnp.float32),
            pltpu.VMEM((TAIL, NEW_EDGE), jnp.float32),
            pltpu.VMEM((TAIL * NEW_EDGE // 128, 128), jnp.float32),
            pltpu.SemaphoreType.DMA,
            pltpu.SemaphoreType.DMA,
            pltpu.SemaphoreType.DMA,
            pltpu.SemaphoreType.DMA,
        ],
    )
    def body(src_hbm, dst_hbm, ps_hbm, pd_hbm, t_hbm, out_hbm,
             sidx, didx, srows, drows, trows, zbuf, agg,
             tsidx, tdidx, tsrows, tdrows, ttrows,
             sem_i0, sem_i1, sem_g0, sem_g1):
        c = lax.axis_index("c")
        s = lax.axis_index("s")
        wid = s * NC + c
        sem_i = (sem_i0, sem_i1)
        sem_g = (sem_g0, sem_g1)

        @pl.when(s < COPY_SUBS)
        def _():
            def zero_row(j, _):
                zbuf[j] = jnp.zeros((NEW_EDGE,), jnp.float32)
                return 0

            lax.fori_loop(0, ROWS_PER_SUB, zero_row, 0)
            pltpu.sync_copy(zbuf, agg.at[pl.ds(s * ROWS_PER_SUB, ROWS_PER_SUB)])

        plsc.subcore_barrier()

        def issue_idx(ci, b):
            ebase = wid * EW + ci * CHUNK
            pltpu.async_copy(src_hbm.at[pl.ds(ebase, CHUNK)], sidx.at[b],
                             sem_i[b])
            pltpu.async_copy(dst_hbm.at[pl.ds(ebase, CHUNK)], didx.at[b],
                             sem_i[b])
            pltpu.async_copy(
                t_hbm.at[pl.ds(ebase * NEW_EDGE // 128, CHUNK * NEW_EDGE // 128)],
                trows.at[b], sem_i[b])

        def wait_idx(ci, b):
            ebase = wid * EW + ci * CHUNK
            pltpu.make_async_copy(src_hbm.at[pl.ds(ebase, CHUNK)],
                                  sidx.at[b], sem_i[b]).wait()
            pltpu.make_async_copy(dst_hbm.at[pl.ds(ebase, CHUNK)],
                                  didx.at[b], sem_i[b]).wait()
            pltpu.make_async_copy(
                t_hbm.at[pl.ds(ebase * NEW_EDGE // 128, CHUNK * NEW_EDGE // 128)],
                trows.at[b], sem_i[b]).wait()

        def issue_gather(b):
            pltpu.async_copy(ps_hbm.at[sidx.at[b]], srows.at[b], sem_g[b])
            pltpu.async_copy(pd_hbm.at[didx.at[b]], drows.at[b], sem_g[b])

        def wait_gather(b):
            pltpu.make_async_copy(ps_hbm.at[sidx.at[b]], srows.at[b],
                                  sem_g[b]).wait()
            pltpu.make_async_copy(pd_hbm.at[didx.at[b]], drows.at[b],
                                  sem_g[b]).wait()

        def process(ci, b):
            @pl.when(ci + 1 < NCHUNKS)
            def _():
                wait_idx(ci + 1, 1 - b)
                issue_gather(1 - b)

            wait_gather(b)
            sr = srows.at[b]
            dr = drows.at[b]
            tr = trows.at[b]

            def edge_body(jj, _):
                for k in range(4):
                    j = jj * 4 + k
                    v = sr[j] + dr[j] + tr[j // 8, pl.ds((j % 8) * NEW_EDGE,
                                                         NEW_EDGE)]
                    sr[j] = jnp.maximum(v, 0.0)
                return 0

            lax.fori_loop(0, CHUNK // 4, edge_body, 0)
            pltpu.sync_copy(sr, agg.at[didx.at[b]], add=True)

            @pl.when(ci + 2 < NCHUNKS)
            def _():
                issue_idx(ci + 2, b)

        issue_idx(0, 0)
        issue_idx(1, 1)
        wait_idx(0, 0)
        issue_gather(0)

        def pair_body(i, _):
            ci = 2 * i
            process(ci, 0)
            process(ci + 1, 1)
            return 0

        lax.fori_loop(0, NCHUNKS // 2, pair_body, 0)
        if NCHUNKS % 2:
            process(NCHUNKS - 1, 0)

        if TAIL:
            tbase = wid * EW + NCHUNKS * CHUNK
            pltpu.sync_copy(src_hbm.at[pl.ds(tbase, TAIL)], tsidx)
            pltpu.sync_copy(dst_hbm.at[pl.ds(tbase, TAIL)], tdidx)
            cp1 = pltpu.async_copy(ps_hbm.at[tsidx], tsrows, sem_g0)
            cp2 = pltpu.async_copy(pd_hbm.at[tdidx], tdrows, sem_g1)
            pltpu.sync_copy(
                t_hbm.at[pl.ds(tbase * NEW_EDGE // 128,
                               TAIL * NEW_EDGE // 128)], ttrows)
            cp1.wait()
            cp2.wait()

            def tail_body(j, _):
                v = (tsrows[j] + tdrows[j]
                     + ttrows[j // 8, pl.ds((j % 8) * NEW_EDGE, NEW_EDGE)])
                tsrows[j] = jnp.maximum(v, 0.0)
                return 0

            lax.fori_loop(0, TAIL, tail_body, 0)
            pltpu.sync_copy(tsrows, agg.at[tdidx], add=True)

        plsc.subcore_barrier()

        @pl.when(s < COPY_SUBS)
        def _():
            base = s * ROWS_PER_SUB
            pltpu.sync_copy(
                agg.at[pl.ds(base, ROWS_PER_SUB)],
                out_hbm.at[c, pl.ds(base, ROWS_PER_SUB)],
            )

    return body(src, dst, p_src, p_dst, t8)


def _node_update(agg0, agg1, q, W2b, b2, W3, b3):
    blk = 1000

    def body(a0_ref, a1_ref, q_ref, w2_ref, b2_ref, w3_ref, b3_ref, o_ref):
        agg = a0_ref[...] + a1_ref[...]
        h = jnp.dot(agg, w2_ref[...], preferred_element_type=jnp.float32)
        h = jnp.maximum(h + q_ref[...] + b2_ref[...], 0.0)
        o_ref[...] = (
            jnp.dot(h, w3_ref[...], preferred_element_type=jnp.float32)
            + b3_ref[...]
        )

    return pl.pallas_call(
        body,
        grid=(N // blk,),
        in_specs=[
            pl.BlockSpec((blk, NEW_EDGE), lambda i: (i, 0)),
            pl.BlockSpec((blk, NEW_EDGE), lambda i: (i, 0)),
            pl.BlockSpec((blk, NEW_NODE), lambda i: (i, 0)),
            pl.BlockSpec((NEW_EDGE, NEW_NODE), lambda i: (0, 0)),
            pl.BlockSpec((1, NEW_NODE), lambda i: (0, 0)),
            pl.BlockSpec((NEW_NODE, FINAL_NODE), lambda i: (0, 0)),
            pl.BlockSpec((1, FINAL_NODE), lambda i: (0, 0)),
        ],
        out_specs=pl.BlockSpec((blk, FINAL_NODE), lambda i: (i, 0)),
        out_shape=jax.ShapeDtypeStruct((N, FINAL_NODE), jnp.float32),
    )(agg0, agg1, q, W2b, b2, W3, b3)


def kernel(x, edge_index, edge_attr, W1, b1, W2, b2, W3, b3):
    W1a = W1[:D_NODE]
    W1b = W1[D_NODE:2 * D_NODE]
    W1c = W1[2 * D_NODE:]
    W2a = W2[:D_NODE]
    W2b = W2[D_NODE:]

    edge8 = edge_attr.reshape(E // 8, 128)
    Wbd = jnp.kron(jnp.eye(8, dtype=jnp.float32), W1c)
    b1t = jnp.tile(b1, 8).reshape(1, 128)
    p_src, p_dst, q, t8 = _prologue_proj(x, W1a, W1b, W2a, edge8, Wbd, b1t)

    src = edge_index[0]
    dst = edge_index[1]
    aggs = _edge_aggregate(src, dst, p_src, p_dst, t8)

    out = _node_update(aggs[0], aggs[1], q, W2b, b2.reshape(1, -1),
                       W3, b3.reshape(1, -1))
    return out


# speedup vs baseline: 9.3881x; 1.0832x over previous
import functools

import jax
import jax.numpy as jnp
from jax import lax
from jax.experimental import pallas as pl
from jax.experimental.pallas import tpu as pltpu
from jax.experimental.pallas import tpu_sc as plsc

N = 10000
E = 320000
D_NODE = 128
D_EDGE = 16
NEW_EDGE = 16
NEW_NODE = 8
FINAL_NODE = 4

NC = 2
NS = 16
NW = NC * NS
EW = E // NW
CHUNK = 128
NCHUNKS = EW // CHUNK
TAIL = EW - NCHUNKS * CHUNK
COPY_SUBS = 10
ROWS_PER_SUB = N // COPY_SUBS


def _prologue_proj(x, Wab, W2a, edge8, Wbd, b1t):
    nblk = 1000
    eblk = 2000
    erows = E // 8

    def body(x_ref, wab_ref, wq_ref, a_ref, w_ref, b_ref,
             p2_ref, q_ref, t_ref):
        xb = x_ref[...]
        p2_ref[...] = jnp.dot(xb, wab_ref[0], preferred_element_type=jnp.float32)
        q_ref[...] = jnp.dot(xb, wq_ref[...], preferred_element_type=jnp.float32)
        t_ref[...] = (
            jnp.dot(a_ref[...], w_ref[...], preferred_element_type=jnp.float32)
            + b_ref[...]
        )

    return pl.pallas_call(
        body,
        grid=(2 * N // nblk,),
        in_specs=[
            pl.BlockSpec((nblk, D_NODE), lambda i: (i % 10, 0)),
            pl.BlockSpec((1, D_NODE, NEW_EDGE), lambda i: (i // 10, 0, 0)),
            pl.BlockSpec((D_NODE, NEW_NODE), lambda i: (0, 0)),
            pl.BlockSpec((eblk, 128), lambda i: (i, 0)),
            pl.BlockSpec((128, 128), lambda i: (0, 0)),
            pl.BlockSpec((1, 128), lambda i: (0, 0)),
        ],
        out_specs=[
            pl.BlockSpec((nblk, NEW_EDGE), lambda i: (i, 0)),
            pl.BlockSpec((nblk, NEW_NODE), lambda i: (i % 10, 0)),
            pl.BlockSpec((eblk, 128), lambda i: (i, 0)),
        ],
        out_shape=[
            jax.ShapeDtypeStruct((2 * N, NEW_EDGE), jnp.float32),
            jax.ShapeDtypeStruct((N, NEW_NODE), jnp.float32),
            jax.ShapeDtypeStruct((erows, 128), jnp.float32),
        ],
    )(x, Wab, W2a, edge8, Wbd, b1t)


def _edge_aggregate(src, dst, p2, t8):
    mesh = plsc.VectorSubcoreMesh(core_axis_name="c", subcore_axis_name="s")

    @functools.partial(
        pl.kernel,
        mesh=mesh,
        compiler_params=pltpu.CompilerParams(use_tc_tiling_on_sc=False),
        out_type=jax.ShapeDtypeStruct((NC, N, NEW_EDGE), jnp.float32),
        scratch_types=[
            pltpu.VMEM((2, CHUNK), jnp.int32),
            pltpu.VMEM((2, CHUNK), jnp.int32),
            pltpu.VMEM((2, CHUNK, NEW_EDGE), jnp.float32),
            pltpu.VMEM((2, CHUNK, NEW_EDGE), jnp.float32),
            pltpu.VMEM((2, CHUNK * NEW_EDGE // 128, 128), jnp.float32),
            pltpu.VMEM((ROWS_PER_SUB, NEW_EDGE), jnp.float32),
            pltpu.VMEM_SHARED((N, NEW_EDGE), jnp.float32),
            pltpu.VMEM((TAIL,), jnp.int32),
            pltpu.VMEM((TAIL,), jnp.int32),
            pltpu.VMEM((TAIL, NEW_EDGE), jnp.float32),
            pltpu.VMEM((TAIL, NEW_EDGE), jnp.float32),
            pltpu.VMEM((TAIL * NEW_EDGE // 128, 128), jnp.float32),
            pltpu.SemaphoreType.DMA,
            pltpu.SemaphoreType.DMA,
            pltpu.SemaphoreType.DMA,
            pltpu.SemaphoreType.DMA,
        ],
    )
    def body(src_hbm, dst_hbm, p2_hbm, t_hbm, out_hbm,
             sidx, didx, srows, drows, trows, zbuf, agg,
             tsidx, tdidx, tsrows, tdrows, ttrows,
             sem_i0, sem_i1, sem_g0, sem_g1):
        c = lax.axis_index("c")
        s = lax.axis_index("s")
        wid = s * NC + c
        sem_i = (sem_i0, sem_i1)
        sem_g = (sem_g0, sem_g1)

        @pl.when(s < COPY_SUBS)
        def _():
            def zero_row(j, _):
                zbuf[j] = jnp.zeros((NEW_EDGE,), jnp.float32)
                return 0

            lax.fori_loop(0, ROWS_PER_SUB, zero_row, 0)
            pltpu.sync_copy(zbuf, agg.at[pl.ds(s * ROWS_PER_SUB, ROWS_PER_SUB)])

        plsc.subcore_barrier()

        def issue_idx(ci, b):
            ebase = wid * EW + ci * CHUNK
            pltpu.async_copy(src_hbm.at[pl.ds(ebase, CHUNK)], sidx.at[b],
                             sem_i[b])
            pltpu.async_copy(dst_hbm.at[pl.ds(ebase, CHUNK)], didx.at[b],
                             sem_i[b])
            pltpu.async_copy(
                t_hbm.at[pl.ds(ebase * NEW_EDGE // 128, CHUNK * NEW_EDGE // 128)],
                trows.at[b], sem_i[b])

        def wait_idx(ci, b):
            ebase = wid * EW + ci * CHUNK
            pltpu.make_async_copy(src_hbm.at[pl.ds(ebase, CHUNK)],
                                  sidx.at[b], sem_i[b]).wait()
            pltpu.make_async_copy(dst_hbm.at[pl.ds(ebase, CHUNK)],
                                  didx.at[b], sem_i[b]).wait()
            pltpu.make_async_copy(
                t_hbm.at[pl.ds(ebase * NEW_EDGE // 128, CHUNK * NEW_EDGE // 128)],
                trows.at[b], sem_i[b]).wait()

        def shift_src(b):
            six = sidx.at[b]
            for k in range(CHUNK // 16):
                six[pl.ds(k * 16, 16)] = six[pl.ds(k * 16, 16)] + N

        def issue_gather(b):
            pltpu.async_copy(p2_hbm.at[sidx.at[b]], srows.at[b], sem_g[b])
            pltpu.async_copy(p2_hbm.at[didx.at[b]], drows.at[b], sem_g[b])

        def wait_gather(b):
            pltpu.make_async_copy(p2_hbm.at[sidx.at[b]], srows.at[b],
                                  sem_g[b]).wait()
            pltpu.make_async_copy(p2_hbm.at[didx.at[b]], drows.at[b],
                                  sem_g[b]).wait()

        def process(ci, b):
            @pl.when(ci + 1 < NCHUNKS)
            def _():
                wait_idx(ci + 1, 1 - b)
                shift_src(1 - b)
                issue_gather(1 - b)

            wait_gather(b)
            sr = srows.at[b]
            dr = drows.at[b]
            tr = trows.at[b]

            def edge_body(jj, _):
                for k in range(8):
                    j = jj * 8 + k
                    v = sr[j] + dr[j] + tr[j // 8, pl.ds((j % 8) * NEW_EDGE,
                                                         NEW_EDGE)]
                    sr[j] = jnp.maximum(v, 0.0)
                return 0

            lax.fori_loop(0, CHUNK // 8, edge_body, 0)
            pltpu.sync_copy(sr, agg.at[didx.at[b]], add=True)

            @pl.when(ci + 2 < NCHUNKS)
            def _():
                issue_idx(ci + 2, b)

        issue_idx(0, 0)
        issue_idx(1, 1)
        wait_idx(0, 0)
        shift_src(0)
        issue_gather(0)

        def pair_body(i, _):
            ci = 2 * i
            process(ci, 0)
            process(ci + 1, 1)
            return 0

        lax.fori_loop(0, NCHUNKS // 2, pair_body, 0)
        if NCHUNKS % 2:
            process(NCHUNKS - 1, 0)

        if TAIL:
            tbase = wid * EW + NCHUNKS * CHUNK
            pltpu.sync_copy(src_hbm.at[pl.ds(tbase, TAIL)], tsidx)
            pltpu.sync_copy(dst_hbm.at[pl.ds(tbase, TAIL)], tdidx)
            for k in range(TAIL // 16):
                tsidx[pl.ds(k * 16, 16)] = tsidx[pl.ds(k * 16, 16)] + N
            cp1 = pltpu.async_copy(p2_hbm.at[tsidx], tsrows, sem_g0)
            cp2 = pltpu.async_copy(p2_hbm.at[tdidx], tdrows, sem_g1)
            pltpu.sync_copy(
                t_hbm.at[pl.ds(tbase * NEW_EDGE // 128,
                               TAIL * NEW_EDGE // 128)], ttrows)
            cp1.wait()
            cp2.wait()

            def tail_body(j, _):
                v = (tsrows[j] + tdrows[j]
                     + ttrows[j // 8, pl.ds((j % 8) * NEW_EDGE, NEW_EDGE)])
                tsrows[j] = jnp.maximum(v, 0.0)
                return 0

            lax.fori_loop(0, TAIL, tail_body, 0)
            pltpu.sync_copy(tsrows, agg.at[tdidx], add=True)

        plsc.subcore_barrier()

        @pl.when(s < COPY_SUBS)
        def _():
            base = s * ROWS_PER_SUB
            pltpu.sync_copy(
                agg.at[pl.ds(base, ROWS_PER_SUB)],
                out_hbm.at[c, pl.ds(base, ROWS_PER_SUB)],
            )

    return body(src, dst, p2, t8)


def _node_update(agg0, agg1, q, W2b, b2, W3, b3):
    blk = 1000

    def body(a0_ref, a1_ref, q_ref, w2_ref, b2_ref, w3_ref, b3_ref, o_ref):
        agg = a0_ref[...] + a1_ref[...]
        h = jnp.dot(agg, w2_ref[...], preferred_element_type=jnp.float32)
        h = jnp.maximum(h + q_ref[...] + b2_ref[...], 0.0)
        o_ref[...] = (
            jnp.dot(h, w3_ref[...], preferred_element_type=jnp.float32)
            + b3_ref[...]
        )

    return pl.pallas_call(
        body,
        grid=(N // blk,),
        in_specs=[
            pl.BlockSpec((blk, NEW_EDGE), lambda i: (i, 0)),
            pl.BlockSpec((blk, NEW_EDGE), lambda i: (i, 0)),
            pl.BlockSpec((blk, NEW_NODE), lambda i: (i, 0)),
            pl.BlockSpec((NEW_EDGE, NEW_NODE), lambda i: (0, 0)),
            pl.BlockSpec((1, NEW_NODE), lambda i: (0, 0)),
            pl.BlockSpec((NEW_NODE, FINAL_NODE), lambda i: (0, 0)),
            pl.BlockSpec((1, FINAL_NODE), lambda i: (0, 0)),
        ],
        out_specs=pl.BlockSpec((blk, FINAL_NODE), lambda i: (i, 0)),
        out_shape=jax.ShapeDtypeStruct((N, FINAL_NODE), jnp.float32),
    )(agg0, agg1, q, W2b, b2, W3, b3)


def kernel(x, edge_index, edge_attr, W1, b1, W2, b2, W3, b3):
    W1a = W1[:D_NODE]
    W1b = W1[D_NODE:2 * D_NODE]
    W1c = W1[2 * D_NODE:]
    W2a = W2[:D_NODE]
    W2b = W2[D_NODE:]

    edge8 = edge_attr.reshape(E // 8, 128)
    Wbd = jnp.kron(jnp.eye(8, dtype=jnp.float32), W1c)
    b1t = jnp.tile(b1, 8).reshape(1, 128)
    Wab = jnp.stack([W1b, W1a])
    p2, q, t8 = _prologue_proj(x, Wab, W2a, edge8, Wbd, b1t)

    src = edge_index[0]
    dst = edge_index[1]
    aggs = _edge_aggregate(src, dst, p2, t8)

    out = _node_update(aggs[0], aggs[1], q, W2b, b2.reshape(1, -1),
                       W3, b3.reshape(1, -1))
    return out


# speedup vs baseline: 9.7274x; 1.0361x over previous
import functools

import jax
import jax.numpy as jnp
from jax import lax
from jax.experimental import pallas as pl
from jax.experimental.pallas import tpu as pltpu
from jax.experimental.pallas import tpu_sc as plsc

N = 10000
E = 320000
D_NODE = 128
D_EDGE = 16
NEW_EDGE = 16
NEW_NODE = 8
FINAL_NODE = 4

NC = 2
NS = 16
NW = NC * NS
EW = E // NW
CHUNK = 128
NCHUNKS = EW // CHUNK
TAIL = EW - NCHUNKS * CHUNK
COPY_SUBS = 10
ROWS_PER_SUB = N // COPY_SUBS


def _prologue_proj(x, Wab, W2a, edge8, Wbd, b1t):
    nblk = 1000
    eblk = 2000
    erows = E // 8

    def body(x_ref, wab_ref, wq_ref, a_ref, w_ref, b_ref,
             p2_ref, q_ref, t_ref):
        xb = x_ref[...]
        p2_ref[...] = jnp.dot(xb, wab_ref[0], preferred_element_type=jnp.float32)
        q_ref[...] = jnp.dot(xb, wq_ref[...], preferred_element_type=jnp.float32)
        t_ref[...] = (
            jnp.dot(a_ref[...], w_ref[...], preferred_element_type=jnp.float32)
            + b_ref[...]
        )

    return pl.pallas_call(
        body,
        grid=(2 * N // nblk,),
        in_specs=[
            pl.BlockSpec((nblk, D_NODE), lambda i: (i % 10, 0)),
            pl.BlockSpec((1, D_NODE, NEW_EDGE), lambda i: (i // 10, 0, 0)),
            pl.BlockSpec((D_NODE, NEW_NODE), lambda i: (0, 0)),
            pl.BlockSpec((eblk, 128), lambda i: (i, 0)),
            pl.BlockSpec((128, 128), lambda i: (0, 0)),
            pl.BlockSpec((1, 128), lambda i: (0, 0)),
        ],
        out_specs=[
            pl.BlockSpec((nblk, NEW_EDGE), lambda i: (i, 0)),
            pl.BlockSpec((nblk, NEW_NODE), lambda i: (i % 10, 0)),
            pl.BlockSpec((eblk, 128), lambda i: (i, 0)),
        ],
        out_shape=[
            jax.ShapeDtypeStruct((2 * N, NEW_EDGE), jnp.float32),
            jax.ShapeDtypeStruct((N, NEW_NODE), jnp.float32),
            jax.ShapeDtypeStruct((erows, 128), jnp.float32),
        ],
    )(x, Wab, W2a, edge8, Wbd, b1t)


def _edge_aggregate(src, dst, p2, t8):
    mesh = plsc.VectorSubcoreMesh(core_axis_name="c", subcore_axis_name="s")

    @functools.partial(
        pl.kernel,
        mesh=mesh,
        compiler_params=pltpu.CompilerParams(use_tc_tiling_on_sc=False),
        out_type=jax.ShapeDtypeStruct((NC, N, NEW_EDGE), jnp.float32),
        scratch_types=[
            pltpu.VMEM((2, CHUNK), jnp.int32),
            pltpu.VMEM((2, CHUNK), jnp.int32),
            pltpu.VMEM((2, CHUNK, NEW_EDGE), jnp.float32),
            pltpu.VMEM((2, CHUNK, NEW_EDGE), jnp.float32),
            pltpu.VMEM((2, CHUNK * NEW_EDGE // 128, 128), jnp.float32),
            pltpu.VMEM((ROWS_PER_SUB, NEW_EDGE), jnp.float32),
            pltpu.VMEM_SHARED((N, NEW_EDGE), jnp.float32),
            pltpu.VMEM((TAIL,), jnp.int32),
            pltpu.VMEM((TAIL,), jnp.int32),
            pltpu.VMEM((TAIL, NEW_EDGE), jnp.float32),
            pltpu.VMEM((TAIL, NEW_EDGE), jnp.float32),
            pltpu.VMEM((TAIL * NEW_EDGE // 128, 128), jnp.float32),
            pltpu.VMEM((2, CHUNK), jnp.int32),
            pltpu.SemaphoreType.DMA,
            pltpu.SemaphoreType.DMA,
            pltpu.SemaphoreType.DMA,
            pltpu.SemaphoreType.DMA,
            pltpu.SemaphoreType.DMA,
            pltpu.SemaphoreType.DMA,
        ],
    )
    def body(src_hbm, dst_hbm, p2_hbm, t_hbm, out_hbm,
             sidx, didx, srows, drows, trows, zbuf, agg,
             tsidx, tdidx, tsrows, tdrows, ttrows, dscat,
             sem_i0, sem_i1, sem_g0, sem_g1, sem_s0, sem_s1):
        c = lax.axis_index("c")
        s = lax.axis_index("s")
        wid = s * NC + c
        sem_i = (sem_i0, sem_i1)
        sem_g = (sem_g0, sem_g1)
        sem_s = (sem_s0, sem_s1)

        @pl.when(s < COPY_SUBS)
        def _():
            def zero_row(jj, _):
                for k in range(8):
                    zbuf[jj * 8 + k] = jnp.zeros((NEW_EDGE,), jnp.float32)
                return 0

            lax.fori_loop(0, ROWS_PER_SUB // 8, zero_row, 0)
            pltpu.sync_copy(zbuf, agg.at[pl.ds(s * ROWS_PER_SUB, ROWS_PER_SUB)])

        plsc.subcore_barrier()

        def issue_idx(ci, b):
            ebase = wid * EW + ci * CHUNK
            pltpu.async_copy(src_hbm.at[pl.ds(ebase, CHUNK)], sidx.at[b],
                             sem_i[b])
            pltpu.async_copy(dst_hbm.at[pl.ds(ebase, CHUNK)], didx.at[b],
                             sem_i[b])
            pltpu.async_copy(
                t_hbm.at[pl.ds(ebase * NEW_EDGE // 128, CHUNK * NEW_EDGE // 128)],
                trows.at[b], sem_i[b])

        def wait_idx(ci, b):
            ebase = wid * EW + ci * CHUNK
            pltpu.make_async_copy(src_hbm.at[pl.ds(ebase, CHUNK)],
                                  sidx.at[b], sem_i[b]).wait()
            pltpu.make_async_copy(dst_hbm.at[pl.ds(ebase, CHUNK)],
                                  didx.at[b], sem_i[b]).wait()
            pltpu.make_async_copy(
                t_hbm.at[pl.ds(ebase * NEW_EDGE // 128, CHUNK * NEW_EDGE // 128)],
                trows.at[b], sem_i[b]).wait()

        def shift_src(b):
            six = sidx.at[b]
            for k in range(CHUNK // 16):
                six[pl.ds(k * 16, 16)] = six[pl.ds(k * 16, 16)] + N

        def issue_gather(b):
            pltpu.async_copy(p2_hbm.at[sidx.at[b]], srows.at[b], sem_g[b])
            pltpu.async_copy(p2_hbm.at[didx.at[b]], drows.at[b], sem_g[b])

        def wait_gather(b):
            pltpu.make_async_copy(p2_hbm.at[sidx.at[b]], srows.at[b],
                                  sem_g[b]).wait()
            pltpu.make_async_copy(p2_hbm.at[didx.at[b]], drows.at[b],
                                  sem_g[b]).wait()

        def wait_scatter(b):
            pltpu.make_async_copy(srows.at[b], agg.at[dscat.at[b]],
                                  sem_s[b]).wait()

        def process(ci, b):
            @pl.when(ci + 1 < NCHUNKS)
            def _():
                wait_idx(ci + 1, 1 - b)
                shift_src(1 - b)

                @pl.when(ci >= 1)
                def _():
                    wait_scatter(1 - b)

                issue_gather(1 - b)

            wait_gather(b)
            sr = srows.at[b]
            dr = drows.at[b]
            tr = trows.at[b]
            dix = didx.at[b]
            dsc = dscat.at[b]

            def edge_body(jj, _):
                for k in range(8):
                    j = jj * 8 + k
                    v = sr[j] + dr[j] + tr[j // 8, pl.ds((j % 8) * NEW_EDGE,
                                                         NEW_EDGE)]
                    sr[j] = jnp.maximum(v, 0.0)
                return 0

            lax.fori_loop(0, CHUNK // 8, edge_body, 0)
            for k in range(CHUNK // 16):
                dsc[pl.ds(k * 16, 16)] = dix[pl.ds(k * 16, 16)]
            pltpu.async_copy(sr, agg.at[dsc], sem_s[b], add=True)

            @pl.when(ci + 2 < NCHUNKS)
            def _():
                issue_idx(ci + 2, b)

        issue_idx(0, 0)
        issue_idx(1, 1)
        wait_idx(0, 0)
        shift_src(0)
        issue_gather(0)

        def pair_body(i, _):
            ci = 2 * i
            process(ci, 0)
            process(ci + 1, 1)
            return 0

        lax.fori_loop(0, NCHUNKS // 2, pair_body, 0)
        if NCHUNKS % 2:
            process(NCHUNKS - 1, 0)
        wait_scatter(0)
        wait_scatter(1)

        if TAIL:
            tbase = wid * EW + NCHUNKS * CHUNK
            pltpu.sync_copy(src_hbm.at[pl.ds(tbase, TAIL)], tsidx)
            pltpu.sync_copy(dst_hbm.at[pl.ds(tbase, TAIL)], tdidx)
            for k in range(TAIL // 16):
                tsidx[pl.ds(k * 16, 16)] = tsidx[pl.ds(k * 16, 16)] + N
            cp1 = pltpu.async_copy(p2_hbm.at[tsidx], tsrows, sem_g0)
            cp2 = pltpu.async_copy(p2_hbm.at[tdidx], tdrows, sem_g1)
            pltpu.sync_copy(
                t_hbm.at[pl.ds(tbase * NEW_EDGE // 128,
                               TAIL * NEW_EDGE // 128)], ttrows)
            cp1.wait()
            cp2.wait()

            def tail_body(j, _):
                v = (tsrows[j] + tdrows[j]
                     + ttrows[j // 8, pl.ds((j % 8) * NEW_EDGE, NEW_EDGE)])
                tsrows[j] = jnp.maximum(v, 0.0)
                return 0

            lax.fori_loop(0, TAIL, tail_body, 0)
            pltpu.sync_copy(tsrows, agg.at[tdidx], add=True)

        plsc.subcore_barrier()

        @pl.when(s < COPY_SUBS)
        def _():
            base = s * ROWS_PER_SUB
            pltpu.sync_copy(
                agg.at[pl.ds(base, ROWS_PER_SUB)],
                out_hbm.at[c, pl.ds(base, ROWS_PER_SUB)],
            )

    return body(src, dst, p2, t8)


def _node_update(agg0, agg1, q, W2b, b2, W3, b3):
    blk = 1000

    def body(a0_ref, a1_ref, q_ref, w2_ref, b2_ref, w3_ref, b3_ref, o_ref):
        agg = a0_ref[...] + a1_ref[...]
        h = jnp.dot(agg, w2_ref[...], preferred_element_type=jnp.float32)
        h = jnp.maximum(h + q_ref[...] + b2_ref[...], 0.0)
        o_ref[...] = (
            jnp.dot(h, w3_ref[...], preferred_element_type=jnp.float32)
            + b3_ref[...]
        )

    return pl.pallas_call(
        body,
        grid=(N // blk,),
        in_specs=[
            pl.BlockSpec((blk, NEW_EDGE), lambda i: (i, 0)),
            pl.BlockSpec((blk, NEW_EDGE), lambda i: (i, 0)),
            pl.BlockSpec((blk, NEW_NODE), lambda i: (i, 0)),
            pl.BlockSpec((NEW_EDGE, NEW_NODE), lambda i: (0, 0)),
            pl.BlockSpec((1, NEW_NODE), lambda i: (0, 0)),
            pl.BlockSpec((NEW_NODE, FINAL_NODE), lambda i: (0, 0)),
            pl.BlockSpec((1, FINAL_NODE), lambda i: (0, 0)),
        ],
        out_specs=pl.BlockSpec((blk, FINAL_NODE), lambda i: (i, 0)),
        out_shape=jax.ShapeDtypeStruct((N, FINAL_NODE), jnp.float32),
    )(agg0, agg1, q, W2b, b2, W3, b3)


def kernel(x, edge_index, edge_attr, W1, b1, W2, b2, W3, b3):
    W1a = W1[:D_NODE]
    W1b = W1[D_NODE:2 * D_NODE]
    W1c = W1[2 * D_NODE:]
    W2a = W2[:D_NODE]
    W2b = W2[D_NODE:]

    edge8 = edge_attr.reshape(E // 8, 128)
    Wbd = jnp.kron(jnp.eye(8, dtype=jnp.float32), W1c)
    b1t = jnp.tile(b1, 8).reshape(1, 128)
    Wab = jnp.stack([W1b, W1a])
    p2, q, t8 = _prologue_proj(x, Wab, W2a, edge8, Wbd, b1t)

    src = edge_index[0]
    dst = edge_index[1]
    aggs = _edge_aggregate(src, dst, p2, t8)

    out = _node_update(aggs[0], aggs[1], q, W2b, b2.reshape(1, -1),
                       W3, b3.reshape(1, -1))
    return out
